# baseline (speedup 1.0000x reference)
import functools

import jax
import jax.numpy as jnp
from jax import lax
from jax.experimental import pallas as pl
from jax.experimental.pallas import tpu as pltpu
from jax.experimental.pallas import tpu_sc as plsc

N = 100000
K = 10
B = 4
C = 3
CH = 400
NCH = N // CH
JPC = CH // 16
HALF = NCH // 2
CW = K * CH
SCH = 2000
NST = N // SCH
CCH = 2000
NCC = N // CCH

_info = plsc.get_sparse_core_info()
_NC = _info.num_cores
_NS = _info.num_subcores


def _sub_body(a_ref, b_ref, o_ref):
    o_ref[...] = a_ref[...] - b_ref[...]


def _diff(out, target):
    a = out.reshape(1200, 1000)
    b = target.reshape(1200, 1000)
    d = pl.pallas_call(
        _sub_body,
        out_shape=jax.ShapeDtypeStruct((1200, 1000), jnp.float32),
        grid=(10,),
        in_specs=[
            pl.BlockSpec((120, 1000), lambda i: (i, 0)),
            pl.BlockSpec((120, 1000), lambda i: (i, 0)),
        ],
        out_specs=pl.BlockSpec((120, 1000), lambda i: (i, 0)),
    )(a, b)
    return d.reshape(B * N * C)


def _sc_body(df, idxf, wf, lossf, planes, plane, sb0, sb1, i0, i1, w0, w1,
             o0, o1, ss0, ss1, si0, si1, sw0, sw1, so0, so1):
    ci = lax.axis_index("c")
    si = lax.axis_index("s")
    sbufs, ssems = (sb0, sb1), (ss0, ss1)
    ibufs, wbufs, obufs = (i0, i1), (w0, w1), (o0, o1)
    isems, wsems, osems = (si0, si1), (sw0, sw1), (so0, so1)
    v1 = lax.iota(jnp.int32, 16)
    v3 = v1 * 3
    vK = v1 * K

    @pl.when(si < 2 * C * 2)
    def _ab():
        lp = si // 2
        h = si % 2
        b = 2 * ci + lp // 3
        c = lp % 3
        dbase = b * (3 * N)

        def start_st(s, par):
            pltpu.async_copy(
                df.at[pl.ds(dbase + s * (3 * SCH), 3 * SCH)],
                sbufs[par], ssems[par])

        start_st(0, 0)

        def stage2(s2, carry):
            for par in range(2):
                s = s2 * 2 + par

                @pl.when(s + 1 < NST)
                def _pf():
                    start_st(s + 1, 1 - par)

                pltpu.make_async_copy(
                    df.at[pl.ds(dbase + s * (3 * SCH), 3 * SCH)],
                    sbufs[par], ssems[par]).wait()

                def sgrp(j, inner):
                    g = plsc.load_gather(sbufs[par], [v3 + (j * 48 + c)])
                    plane[pl.ds(s * SCH + j * 16, 16)] = g
                    return inner

                lax.fori_loop(0, SCH // 16, sgrp, 0)
            return carry

        lax.fori_loop(0, NST // 2, stage2, 0)

        c0 = h * HALF

        def start_in(cb, par):
            pltpu.async_copy(idxf.at[pl.ds(cb * CW, CW)], ibufs[par], isems[par])
            pltpu.async_copy(wf.at[pl.ds(cb * CW, CW)], wbufs[par], wsems[par])

        start_in(c0, 0)

        def do_chunk(cb2, par):
            cb = c0 + cb2 * 2 + par

            @pl.when(cb + 1 < c0 + HALF)
            def _prefetch():
                start_in(cb + 1, 1 - par)

            pltpu.make_async_copy(
                idxf.at[pl.ds(cb * CW, CW)], ibufs[par], isems[par]).wait()
            pltpu.make_async_copy(
                wf.at[pl.ds(cb * CW, CW)], wbufs[par], wsems[par]).wait()

            @pl.when(cb2 > 0)
            def _reclaim():
                pltpu.make_async_copy(
                    obufs[par], planes.at[pl.ds(0, CH)], osems[par]).wait()

            def grp(j, inner):
                r0 = j * 16
                base = j * (16 * K)
                acc = jnp.zeros((16,), jnp.float32)
                for k in range(K):
                    sel = vK + (base + k)
                    ii = plsc.load_gather(ibufs[par], [sel])
                    g = plsc.load_gather(plane, [ii])
                    ww = plsc.load_gather(wbufs[par], [sel])
                    acc = acc + g * ww
                ctr = plane[pl.ds(cb * CH + r0, 16)]
                v = ctr + acc
                obufs[par][pl.ds(r0, 16)] = v * v
                return inner

            lax.fori_loop(0, JPC, grp, 0)
            pltpu.async_copy(
                obufs[par],
                planes.at[pl.ds((b * 3 + c) * N + cb * CH, CH)], osems[par])

        def chunk2(cb2, carry):
            for par in range(2):
                do_chunk(cb2, par)
            return carry

        lax.fori_loop(0, HALF // 2, chunk2, 0)
        if HALF % 2:
            do_chunk(HALF // 2, 0)
        for par in range(2):
            pltpu.make_async_copy(
                obufs[par], planes.at[pl.ds(0, CH)], osems[par]).wait()

    plsc.subcore_barrier()

    t8 = si % 8
    bl = si // 8
    b2 = 2 * ci + bl
    nch = jnp.where(t8 < 2, (NCC // 8) + 1, NCC // 8)
    start = t8 * (NCC // 8) + jnp.minimum(t8, NCC % 8)

    def cchunk(q, carry):
        r0 = (start + q) * CCH
        for c3 in range(3):
            pltpu.sync_copy(
                planes.at[pl.ds((b2 * 3 + c3) * N + r0, CCH)],
                sb0.at[pl.ds(c3 * CCH, CCH)])

        def cgrp(j, inner):
            for c3 in range(3):
                x = sb0[pl.ds(c3 * CCH + j * 16, 16)]
                plsc.store_scatter(sb1, [v3 + (j * 48 + c3)], x)
            return inner

        lax.fori_loop(0, CCH // 16, cgrp, 0)
        pltpu.sync_copy(sb1, lossf.at[pl.ds(b2 * (3 * N) + r0 * 3, 3 * CCH)])
        return carry

    lax.fori_loop(0, nch, cchunk, 0)


_sc_kernel = functools.partial(
    pl.kernel,
    mesh=plsc.VectorSubcoreMesh(core_axis_name="c", subcore_axis_name="s"),
    compiler_params=pltpu.CompilerParams(needs_layout_passes=False),
    out_type=(
        jax.ShapeDtypeStruct((B * N * C,), jnp.float32),
        jax.ShapeDtypeStruct((B * C * N,), jnp.float32),
    ),
    scratch_types=[
        pltpu.VMEM((N,), jnp.float32),
        pltpu.VMEM((3 * SCH,), jnp.float32),
        pltpu.VMEM((3 * SCH,), jnp.float32),
        pltpu.VMEM((CW,), jnp.int32),
        pltpu.VMEM((CW,), jnp.int32),
        pltpu.VMEM((CW,), jnp.float32),
        pltpu.VMEM((CW,), jnp.float32),
        pltpu.VMEM((CH,), jnp.float32),
        pltpu.VMEM((CH,), jnp.float32),
        pltpu.SemaphoreType.DMA,
        pltpu.SemaphoreType.DMA,
        pltpu.SemaphoreType.DMA,
        pltpu.SemaphoreType.DMA,
        pltpu.SemaphoreType.DMA,
        pltpu.SemaphoreType.DMA,
        pltpu.SemaphoreType.DMA,
        pltpu.SemaphoreType.DMA,
    ],
)(_sc_body)


def kernel(out, target, neighbor_idxs, neighbor_weights):
    idxf = neighbor_idxs.astype(jnp.int32).reshape(-1)
    wf = neighbor_weights.reshape(-1)
    df = _diff(out, target)
    lossf, _ = _sc_kernel(df, idxf, wf)
    return lossf.reshape(B, N, C)
